# Initial kernel scaffold; baseline (speedup 1.0000x reference)
import jax
import jax.numpy as jnp
from jax.experimental import pallas as pl


def kernel(autoregressive_embedding, action_type, entity_embeddings, entity_num, func_W, func_b, conv_W, conv_b, fc1_W, fc1_b, fc2_W, fc2_b, Wih, Whh, bih, bhh, proj_W, proj_b, new_var):
    raise NotImplementedError("write your pallas kernel here")


# speedup vs baseline: 1.9453x; 1.9453x over previous
import jax
import jax.numpy as jnp
from jax.experimental import pallas as pl
from jax.experimental.pallas import tpu as pltpu

_B = 128
_N = 512
_EMB = 256
_MT = 256
_AE = 1024
_O256 = 256
_O32 = 32
_MAXSEL = 64
_NEG = -1.0e9


def _conv_kernel(w_ref, ent_ref, b_ref, out_ref):
    out_ref[...] = jax.lax.dot_general(
        w_ref[...], ent_ref[...], (((1,), (1,)), ((), ())),
        preferred_element_type=jnp.float32) + b_ref[...]


def _dot_t(a, b):
    return jax.lax.dot_general(a, b, (((1,), (1,)), ((), ())),
                               preferred_element_type=jnp.float32)


def _loop_kernel(keyraw, noise, fe, ae0, en, nv, fc1w, fc1b, fc2w, fc2b,
                 wih, whh, bih, bhh, projw, projb,
                 ul, un, aeo,
                 key_s, kavg_s, mask_s, h_s, c_s, ae_s, gate_s):
    i = pl.program_id(0)
    en_col = en[:, 0:1]
    niota = jax.lax.broadcasted_iota(jnp.int32, (_B, _N), 1)

    @pl.when(i == 0)
    def _init():
        valid = niota <= en_col
        cols = []
        for k in range(_O32):
            nv_k = nv[:, k:k + 1]
            k1 = jnp.where(niota == en_col, nv_k,
                           jnp.where(niota == (_N - 1), 0.0, keyraw[k]))
            key_s[k] = k1
            kv = jnp.where(valid, k1, 0.0)
            cols.append(jnp.sum(kv, axis=1, keepdims=True))
        ksum = jnp.concatenate(cols, axis=1)
        kavg_s[...] = ksum / en_col.astype(jnp.float32)
        mask_s[...] = jnp.where(valid & (niota != en_col), 1, 0)
        h_s[...] = jnp.zeros_like(h_s)
        c_s[...] = jnp.zeros_like(c_s)
        ae_s[...] = ae0[...]
        gate_s[...] = jnp.ones_like(gate_s)

    @pl.when(i == 1)
    def _reenable_end():
        mask_s[...] = jnp.where(niota == en_col, 1, mask_s[...])

    ae = ae_s[...]
    x = _dot_t(ae, fc1w[...]) + fc1b[...]
    x = jax.nn.relu(x + fe[...])
    u = _dot_t(x, fc2w[...]) + fc2b[...]
    h = h_s[...]
    c = c_s[...]
    g = _dot_t(u, wih[...]) + _dot_t(h, whh[...]) + bih[...] + bhh[...]
    i_g = g[:, 0:_O32]
    f_g = g[:, _O32:2 * _O32]
    g_g = g[:, 2 * _O32:3 * _O32]
    o_g = g[:, 3 * _O32:4 * _O32]
    c = jax.nn.sigmoid(f_g) * c + jax.nn.sigmoid(i_g) * jnp.tanh(g_g)
    h = jax.nn.sigmoid(o_g) * jnp.tanh(c)
    h_s[...] = h
    c_s[...] = c

    y = key_s[0] * h[:, 0:1]
    for k in range(1, _O32):
        y = y + key_s[k] * h[:, k:k + 1]
    logits = jnp.where(mask_s[...] != 0, y, _NEG)
    ul[0] = logits

    z = noise[0] + logits
    m = jnp.max(z, axis=-1, keepdims=True)
    cand = jnp.where(z == m, niota, _N)
    sample = jnp.min(cand, axis=-1, keepdims=True)
    un[0] = jnp.broadcast_to(sample, (_B, 128))

    mask_s[...] = jnp.where(niota == sample, 0, mask_s[...])
    last = (sample == en_col).astype(jnp.float32)
    gate = gate_s[:, 0:1] * (1.0 - last)
    gate_s[...] = jnp.broadcast_to(gate, (_B, 128))

    oh = (niota == sample).astype(jnp.float32)
    cols = []
    for k in range(_O32):
        cols.append(jnp.sum(key_s[k] * oh, axis=1, keepdims=True))
    sel = jnp.concatenate(cols, axis=1)
    sel = sel.astype(jnp.bfloat16).astype(jnp.float32)
    out = sel - kavg_s[...]
    t = _dot_t(out, projw[...]) + projb[...]
    ae_s[...] = ae + t * gate

    @pl.when(i == _MAXSEL - 1)
    def _finish():
        aeo[...] = ae_s[...]


def kernel(autoregressive_embedding, action_type, entity_embeddings, entity_num,
           func_W, func_b, conv_W, conv_b, fc1_W, fc1_b, fc2_W, fc2_b,
           Wih, Whh, bih, bhh, proj_W, proj_b, new_var):
    orig_ae = autoregressive_embedding

    one_hot_types = jnp.ones((_B, _MT), dtype=jnp.float32)
    fe = jax.nn.relu(one_hot_types @ func_W.T + func_b)

    base = jax.random.key(42)
    noise = jnp.stack(
        [jax.random.gumbel(jax.random.fold_in(base, i), (_B, _N), jnp.float32)
         for i in range(_MAXSEL)], axis=0)

    ent2 = entity_embeddings.reshape(_B * _N, _EMB)
    rows = _B * _N // 16
    key2 = pl.pallas_call(
        _conv_kernel,
        grid=(16,),
        in_specs=[pl.BlockSpec((_O32, _EMB), lambda i: (0, 0)),
                  pl.BlockSpec((rows, _EMB), lambda i: (i, 0)),
                  pl.BlockSpec((_O32, 1), lambda i: (0, 0))],
        out_specs=pl.BlockSpec((_O32, rows), lambda i: (0, i)),
        out_shape=jax.ShapeDtypeStruct((_O32, _B * _N), jnp.float32),
    )(conv_W, ent2, conv_b[:, None])
    keyraw = key2.reshape(_O32, _B, _N)

    en2 = jnp.broadcast_to(entity_num.astype(jnp.int32)[:, None], (_B, 128))
    nv2 = jnp.broadcast_to(new_var[None, :], (_B, _O32))

    full2 = lambda shape: pl.BlockSpec(shape, lambda i: (0, 0))
    ul3, un3, ae_out = pl.pallas_call(
        _loop_kernel,
        grid=(_MAXSEL,),
        in_specs=[
            pl.BlockSpec((_O32, _B, _N), lambda i: (0, 0, 0)),
            pl.BlockSpec((1, _B, _N), lambda i: (i, 0, 0)),
            full2((_B, _O256)),
            full2((_B, _AE)),
            full2((_B, 128)),
            full2((_B, _O32)),
            full2((_O256, _AE)),
            full2((1, _O256)),
            full2((_O32, _O256)),
            full2((1, _O32)),
            full2((4 * _O32, _O32)),
            full2((4 * _O32, _O32)),
            full2((1, 4 * _O32)),
            full2((1, 4 * _O32)),
            full2((_AE, _O32)),
            full2((1, _AE)),
        ],
        out_specs=[
            pl.BlockSpec((1, _B, _N), lambda i: (i, 0, 0)),
            pl.BlockSpec((1, _B, 128), lambda i: (i, 0, 0)),
            full2((_B, _AE)),
        ],
        out_shape=[
            jax.ShapeDtypeStruct((_MAXSEL, _B, _N), jnp.float32),
            jax.ShapeDtypeStruct((_MAXSEL, _B, 128), jnp.int32),
            jax.ShapeDtypeStruct((_B, _AE), jnp.float32),
        ],
        scratch_shapes=[
            pltpu.VMEM((_O32, _B, _N), jnp.float32),
            pltpu.VMEM((_B, _O32), jnp.float32),
            pltpu.VMEM((_B, _N), jnp.int32),
            pltpu.VMEM((_B, _O32), jnp.float32),
            pltpu.VMEM((_B, _O32), jnp.float32),
            pltpu.VMEM((_B, _AE), jnp.float32),
            pltpu.VMEM((_B, 128), jnp.float32),
        ],
    )(keyraw, noise, fe, autoregressive_embedding, en2, nv2,
      fc1_W, fc1_b[None, :], fc2_W, fc2_b[None, :], Wih, Whh,
      bih[None, :], bhh[None, :], proj_W, proj_b[None, :])

    ul = jnp.transpose(ul3, (1, 0, 2))
    un = jnp.transpose(un3[:, :, 0])[:, :, None]

    no_sel = action_type[:, 0] == 0
    ul = jnp.where(no_sel[:, None, None], _NEG, ul)
    un = jnp.where(no_sel[:, None, None], _N - 1, un)
    ae = jnp.where(no_sel[:, None], orig_ae, ae_out)
    return ul, un, ae


# speedup vs baseline: 4.2560x; 2.1879x over previous
import numpy as np
import jax
import jax.numpy as jnp
from jax.experimental import pallas as pl
from jax.experimental.pallas import tpu as pltpu

_B = 128
_N = 512
_EMB = 256
_MT = 256
_AE = 1024
_O256 = 256
_O32 = 32
_MAXSEL = 64
_NEG = -1.0e9


def _gumbel_noise():
    base = jax.random.key(42)
    mats = [jax.random.gumbel(jax.random.fold_in(base, i), (_B, _N), jnp.float32)
            for i in range(_MAXSEL)]
    return np.asarray(jnp.stack(mats, axis=0))


_NOISE = _gumbel_noise()


def _conv_kernel(w_ref, ent_ref, b_ref, out_ref):
    out_ref[...] = jax.lax.dot_general(
        w_ref[...], ent_ref[...], (((1,), (1,)), ((), ())),
        preferred_element_type=jnp.float32) + b_ref[...]


def _dot_t(a, b):
    return jax.lax.dot_general(a, b, (((1,), (1,)), ((), ())),
                               preferred_element_type=jnp.float32)


def _loop_kernel(keyraw, noise, fe, ae0, en, nv, fc1w, fc1b, fc2w, fc2b,
                 wih, whh, bih, bhh, projw, projb,
                 ul, un, aeo,
                 key_s, kavg_s, mask_s, h_s, c_s, ae_s, gate_s):
    i = pl.program_id(0)
    en_col = en[:, 0:1]
    niota = jax.lax.broadcasted_iota(jnp.int32, (_B, _N), 1)

    @pl.when(i == 0)
    def _init():
        valid = niota <= en_col
        cols = []
        for k in range(_O32):
            nv_k = nv[:, k:k + 1]
            k1 = jnp.where(niota == en_col, nv_k,
                           jnp.where(niota == (_N - 1), 0.0, keyraw[k]))
            key_s[k] = k1
            kv = jnp.where(valid, k1, 0.0)
            cols.append(jnp.sum(kv, axis=1, keepdims=True))
        ksum = jnp.concatenate(cols, axis=1)
        kavg_s[...] = ksum / en_col.astype(jnp.float32)
        mask_s[...] = jnp.where(valid & (niota != en_col), 1, 0)
        h_s[...] = jnp.zeros_like(h_s)
        c_s[...] = jnp.zeros_like(c_s)
        ae_s[...] = ae0[...]
        gate_s[...] = jnp.ones_like(gate_s)

    @pl.when(i == 1)
    def _reenable_end():
        mask_s[...] = jnp.where(niota == en_col, 1, mask_s[...])

    ae = ae_s[...]
    x = _dot_t(ae, fc1w[...]) + fc1b[...]
    x = jax.nn.relu(x + fe[...])
    u = _dot_t(x, fc2w[...]) + fc2b[...]
    h = h_s[...]
    c = c_s[...]
    g = _dot_t(u, wih[...]) + _dot_t(h, whh[...]) + bih[...] + bhh[...]
    i_g = g[:, 0:_O32]
    f_g = g[:, _O32:2 * _O32]
    g_g = g[:, 2 * _O32:3 * _O32]
    o_g = g[:, 3 * _O32:4 * _O32]
    c = jax.nn.sigmoid(f_g) * c + jax.nn.sigmoid(i_g) * jnp.tanh(g_g)
    h = jax.nn.sigmoid(o_g) * jnp.tanh(c)
    h_s[...] = h
    c_s[...] = c

    y = key_s[0] * h[:, 0:1]
    for k in range(1, _O32):
        y = y + key_s[k] * h[:, k:k + 1]
    logits = jnp.where(mask_s[...] != 0, y, _NEG)
    ul[0] = logits

    z = noise[0] + logits
    m = jnp.max(z, axis=-1, keepdims=True)
    cand = jnp.where(z == m, niota, _N)
    sample = jnp.min(cand, axis=-1, keepdims=True)
    un[0] = jnp.broadcast_to(sample, (_B, 128))

    mask_s[...] = jnp.where(niota == sample, 0, mask_s[...])
    last = (sample == en_col).astype(jnp.float32)
    gate = gate_s[:, 0:1] * (1.0 - last)
    gate_s[...] = jnp.broadcast_to(gate, (_B, 128))

    oh = (niota == sample).astype(jnp.float32)
    cols = []
    for k in range(_O32):
        cols.append(jnp.sum(key_s[k] * oh, axis=1, keepdims=True))
    sel = jnp.concatenate(cols, axis=1)
    sel = sel.astype(jnp.bfloat16).astype(jnp.float32)
    out = sel - kavg_s[...]
    t = _dot_t(out, projw[...]) + projb[...]
    ae_s[...] = ae + t * gate

    @pl.when(i == _MAXSEL - 1)
    def _finish():
        aeo[...] = ae_s[...]


def kernel(autoregressive_embedding, action_type, entity_embeddings, entity_num,
           func_W, func_b, conv_W, conv_b, fc1_W, fc1_b, fc2_W, fc2_b,
           Wih, Whh, bih, bhh, proj_W, proj_b, new_var):
    orig_ae = autoregressive_embedding

    one_hot_types = jnp.ones((_B, _MT), dtype=jnp.float32)
    fe = jax.nn.relu(one_hot_types @ func_W.T + func_b)

    noise = jnp.asarray(_NOISE)

    ent2 = entity_embeddings.reshape(_B * _N, _EMB)
    rows = _B * _N // 16
    key2 = pl.pallas_call(
        _conv_kernel,
        grid=(16,),
        in_specs=[pl.BlockSpec((_O32, _EMB), lambda i: (0, 0)),
                  pl.BlockSpec((rows, _EMB), lambda i: (i, 0)),
                  pl.BlockSpec((_O32, 1), lambda i: (0, 0))],
        out_specs=pl.BlockSpec((_O32, rows), lambda i: (0, i)),
        out_shape=jax.ShapeDtypeStruct((_O32, _B * _N), jnp.float32),
    )(conv_W, ent2, conv_b[:, None])
    keyraw = key2.reshape(_O32, _B, _N)

    en2 = jnp.broadcast_to(entity_num.astype(jnp.int32)[:, None], (_B, 128))
    nv2 = jnp.broadcast_to(new_var[None, :], (_B, _O32))

    full2 = lambda shape: pl.BlockSpec(shape, lambda i: (0, 0))
    ul3, un3, ae_out = pl.pallas_call(
        _loop_kernel,
        grid=(_MAXSEL,),
        in_specs=[
            pl.BlockSpec((_O32, _B, _N), lambda i: (0, 0, 0)),
            pl.BlockSpec((1, _B, _N), lambda i: (i, 0, 0)),
            full2((_B, _O256)),
            full2((_B, _AE)),
            full2((_B, 128)),
            full2((_B, _O32)),
            full2((_O256, _AE)),
            full2((1, _O256)),
            full2((_O32, _O256)),
            full2((1, _O32)),
            full2((4 * _O32, _O32)),
            full2((4 * _O32, _O32)),
            full2((1, 4 * _O32)),
            full2((1, 4 * _O32)),
            full2((_AE, _O32)),
            full2((1, _AE)),
        ],
        out_specs=[
            pl.BlockSpec((1, _B, _N), lambda i: (i, 0, 0)),
            pl.BlockSpec((1, _B, 128), lambda i: (i, 0, 0)),
            full2((_B, _AE)),
        ],
        out_shape=[
            jax.ShapeDtypeStruct((_MAXSEL, _B, _N), jnp.float32),
            jax.ShapeDtypeStruct((_MAXSEL, _B, 128), jnp.int32),
            jax.ShapeDtypeStruct((_B, _AE), jnp.float32),
        ],
        scratch_shapes=[
            pltpu.VMEM((_O32, _B, _N), jnp.float32),
            pltpu.VMEM((_B, _O32), jnp.float32),
            pltpu.VMEM((_B, _N), jnp.int32),
            pltpu.VMEM((_B, _O32), jnp.float32),
            pltpu.VMEM((_B, _O32), jnp.float32),
            pltpu.VMEM((_B, _AE), jnp.float32),
            pltpu.VMEM((_B, 128), jnp.float32),
        ],
    )(keyraw, noise, fe, autoregressive_embedding, en2, nv2,
      fc1_W, fc1_b[None, :], fc2_W, fc2_b[None, :], Wih, Whh,
      bih[None, :], bhh[None, :], proj_W, proj_b[None, :])

    ul = jnp.transpose(ul3, (1, 0, 2))
    un = jnp.transpose(un3[:, :, 0])[:, :, None]

    no_sel = action_type[:, 0] == 0
    ul = jnp.where(no_sel[:, None, None], _NEG, ul)
    un = jnp.where(no_sel[:, None, None], _N - 1, un)
    ae = jnp.where(no_sel[:, None], orig_ae, ae_out)
    return ul, un, ae


# speedup vs baseline: 4.3741x; 1.0278x over previous
import numpy as np
import jax
import jax.numpy as jnp
from jax.experimental import pallas as pl
from jax.experimental.pallas import tpu as pltpu

_B = 128
_N = 512
_EMB = 256
_MT = 256
_AE = 1024
_O256 = 256
_O32 = 32
_MAXSEL = 64
_NEG = -1.0e9


def _gumbel_stack():
    base = jax.random.key(42)
    return jnp.stack(
        [jax.random.gumbel(jax.random.fold_in(base, i), (_B, _N), jnp.float32)
         for i in range(_MAXSEL)], axis=0)


try:
    _NOISE = np.asarray(_gumbel_stack())
except Exception:
    _NOISE = None


def _conv_kernel(w_ref, ent_ref, b_ref, out_ref):
    out_ref[...] = jax.lax.dot_general(
        w_ref[...], ent_ref[...], (((1,), (1,)), ((), ())),
        preferred_element_type=jnp.float32) + b_ref[...]


def _dot_t(a, b):
    return jax.lax.dot_general(a, b, (((1,), (1,)), ((), ())),
                               preferred_element_type=jnp.float32)


def _loop_kernel(keyraw, noise, fe, ae0, en, nv, fc1w, fc1b, fc2w, fc2b,
                 wih, whh, bih, bhh, projw, projb,
                 ul, un, aeo,
                 key_s, kavg_s, mask_s, h_s, c_s, ae_s, gate_s):
    i = pl.program_id(0)
    en_col = en[:, 0:1]
    niota = jax.lax.broadcasted_iota(jnp.int32, (_B, _N), 1)

    @pl.when(i == 0)
    def _init():
        valid = niota <= en_col
        cols = []
        for k in range(_O32):
            nv_k = nv[:, k:k + 1]
            k1 = jnp.where(niota == en_col, nv_k,
                           jnp.where(niota == (_N - 1), 0.0, keyraw[k]))
            key_s[k] = k1
            kv = jnp.where(valid, k1, 0.0)
            cols.append(jnp.sum(kv, axis=1, keepdims=True))
        ksum = jnp.concatenate(cols, axis=1)
        kavg_s[...] = ksum / en_col.astype(jnp.float32)
        mask_s[...] = jnp.where(valid & (niota != en_col), 1, 0)
        h_s[...] = jnp.zeros_like(h_s)
        c_s[...] = jnp.zeros_like(c_s)
        ae_s[...] = ae0[...]
        gate_s[...] = jnp.ones_like(gate_s)

    @pl.when(i == 1)
    def _reenable_end():
        mask_s[...] = jnp.where(niota == en_col, 1, mask_s[...])

    ae = ae_s[...]
    x = _dot_t(ae, fc1w[...]) + fc1b[...]
    x = jax.nn.relu(x + fe[...])
    u = _dot_t(x, fc2w[...]) + fc2b[...]
    h = h_s[...]
    c = c_s[...]
    g = _dot_t(u, wih[...]) + _dot_t(h, whh[...]) + bih[...] + bhh[...]
    i_g = g[:, 0:_O32]
    f_g = g[:, _O32:2 * _O32]
    g_g = g[:, 2 * _O32:3 * _O32]
    o_g = g[:, 3 * _O32:4 * _O32]
    c = jax.nn.sigmoid(f_g) * c + jax.nn.sigmoid(i_g) * jnp.tanh(g_g)
    h = jax.nn.sigmoid(o_g) * jnp.tanh(c)
    h_s[...] = h
    c_s[...] = c

    y = key_s[0] * h[:, 0:1]
    for k in range(1, _O32):
        y = y + key_s[k] * h[:, k:k + 1]
    logits = jnp.where(mask_s[...] != 0, y, _NEG)
    ul[:, i % 8, :] = logits

    z = noise[0] + logits
    m = jnp.max(z, axis=-1, keepdims=True)
    cand = jnp.where(z == m, niota, _N)
    sample = jnp.min(cand, axis=-1, keepdims=True)
    un[0] = jnp.broadcast_to(sample, (_B, 128))

    mask_s[...] = jnp.where(niota == sample, 0, mask_s[...])
    last = (sample == en_col).astype(jnp.float32)
    gate = gate_s[:, 0:1] * (1.0 - last)
    gate_s[...] = jnp.broadcast_to(gate, (_B, 128))

    oh = (niota == sample).astype(jnp.float32)
    cols = []
    for k in range(_O32):
        cols.append(jnp.sum(key_s[k] * oh, axis=1, keepdims=True))
    sel = jnp.concatenate(cols, axis=1)
    sel = sel.astype(jnp.bfloat16).astype(jnp.float32)
    out = sel - kavg_s[...]
    t = _dot_t(out, projw[...]) + projb[...]
    ae_s[...] = ae + t * gate

    @pl.when(i == _MAXSEL - 1)
    def _finish():
        aeo[...] = ae_s[...]


def kernel(autoregressive_embedding, action_type, entity_embeddings, entity_num,
           func_W, func_b, conv_W, conv_b, fc1_W, fc1_b, fc2_W, fc2_b,
           Wih, Whh, bih, bhh, proj_W, proj_b, new_var):
    orig_ae = autoregressive_embedding

    one_hot_types = jnp.ones((_B, _MT), dtype=jnp.float32)
    fe = jax.nn.relu(one_hot_types @ func_W.T + func_b)

    noise = jnp.asarray(_NOISE) if _NOISE is not None else _gumbel_stack()

    ent2 = entity_embeddings.reshape(_B * _N, _EMB)
    rows = _B * _N // 16
    key2 = pl.pallas_call(
        _conv_kernel,
        grid=(16,),
        in_specs=[pl.BlockSpec((_O32, _EMB), lambda i: (0, 0)),
                  pl.BlockSpec((rows, _EMB), lambda i: (i, 0)),
                  pl.BlockSpec((_O32, 1), lambda i: (0, 0))],
        out_specs=pl.BlockSpec((_O32, rows), lambda i: (0, i)),
        out_shape=jax.ShapeDtypeStruct((_O32, _B * _N), jnp.float32),
    )(conv_W, ent2, conv_b[:, None])
    keyraw = key2.reshape(_O32, _B, _N)

    en2 = jnp.broadcast_to(entity_num.astype(jnp.int32)[:, None], (_B, 128))
    nv2 = jnp.broadcast_to(new_var[None, :], (_B, _O32))

    full2 = lambda shape: pl.BlockSpec(shape, lambda i: (0, 0))
    ul3, un3, ae_out = pl.pallas_call(
        _loop_kernel,
        grid=(_MAXSEL,),
        in_specs=[
            pl.BlockSpec((_O32, _B, _N), lambda i: (0, 0, 0)),
            pl.BlockSpec((1, _B, _N), lambda i: (i, 0, 0)),
            full2((_B, _O256)),
            full2((_B, _AE)),
            full2((_B, 128)),
            full2((_B, _O32)),
            full2((_O256, _AE)),
            full2((1, _O256)),
            full2((_O32, _O256)),
            full2((1, _O32)),
            full2((4 * _O32, _O32)),
            full2((4 * _O32, _O32)),
            full2((1, 4 * _O32)),
            full2((1, 4 * _O32)),
            full2((_AE, _O32)),
            full2((1, _AE)),
        ],
        out_specs=[
            pl.BlockSpec((_B, 8, _N), lambda i: (0, i // 8, 0)),
            pl.BlockSpec((1, _B, 128), lambda i: (i, 0, 0)),
            full2((_B, _AE)),
        ],
        out_shape=[
            jax.ShapeDtypeStruct((_B, _MAXSEL, _N), jnp.float32),
            jax.ShapeDtypeStruct((_MAXSEL, _B, 128), jnp.int32),
            jax.ShapeDtypeStruct((_B, _AE), jnp.float32),
        ],
        scratch_shapes=[
            pltpu.VMEM((_O32, _B, _N), jnp.float32),
            pltpu.VMEM((_B, _O32), jnp.float32),
            pltpu.VMEM((_B, _N), jnp.int32),
            pltpu.VMEM((_B, _O32), jnp.float32),
            pltpu.VMEM((_B, _O32), jnp.float32),
            pltpu.VMEM((_B, _AE), jnp.float32),
            pltpu.VMEM((_B, 128), jnp.float32),
        ],
    )(keyraw, noise, fe, autoregressive_embedding, en2, nv2,
      fc1_W, fc1_b[None, :], fc2_W, fc2_b[None, :], Wih, Whh,
      bih[None, :], bhh[None, :], proj_W, proj_b[None, :])

    ul = ul3
    un = jnp.transpose(un3[:, :, 0])[:, :, None]

    no_sel = action_type[:, 0] == 0
    ul = jnp.where(no_sel[:, None, None], _NEG, ul)
    un = jnp.where(no_sel[:, None, None], _N - 1, un)
    ae = jnp.where(no_sel[:, None], orig_ae, ae_out)
    return ul, un, ae
